# baseline (speedup 1.0000x reference)
import functools

import jax
import jax.numpy as jnp
from jax import lax
from jax.experimental import pallas as pl
from jax.experimental.pallas import tpu as pltpu
from jax.experimental.pallas import tpu_sc as plsc

_DIM = 4096
_E = 64
_K = 8
_BLOCK = 1024

_info = plsc.get_sparse_core_info()
_NC, _NS, _L = _info.num_cores, _info.num_subcores, _info.num_lanes
_NW = _NC * _NS


def _score_block(x_ref, w_ref, keys_ref):
    x = x_ref[...]
    w = w_ref[...]
    scores = jax.lax.dot_general(
        x, w, (((1,), (1,)), ((), ())),
        preferred_element_type=jnp.float32)
    m = jnp.max(scores, axis=1, keepdims=True)
    e = jnp.exp(scores - m)
    p = e / jnp.sum(e, axis=1, keepdims=True)
    lane = jax.lax.broadcasted_iota(jnp.int32, p.shape, 1)
    bits = jax.lax.bitcast_convert_type(p, jnp.int32)
    keys_ref[...] = jax.lax.bitcast_convert_type(
        (bits & ~(_E - 1)) | ((_E - 1) - lane), jnp.float32)


def _tc_scores(x, weight):
    n_tokens = x.shape[0]
    return pl.pallas_call(
        _score_block,
        grid=(n_tokens // _BLOCK,),
        in_specs=[
            pl.BlockSpec((_BLOCK, _DIM), lambda i: (i, 0)),
            pl.BlockSpec((_E, _DIM), lambda i: (0, 0)),
        ],
        out_specs=pl.BlockSpec((_BLOCK, _E), lambda i: (i, 0)),
        out_shape=jax.ShapeDtypeStruct((n_tokens, _E), jnp.float32),
        compiler_params=pltpu.CompilerParams(
            dimension_semantics=("parallel",)),
    )(x, weight)


def _make_sc_topk(n_tokens):
    chunk = n_tokens // _NW
    mesh = plsc.VectorSubcoreMesh(core_axis_name="c", subcore_axis_name="s")

    @functools.partial(
        pl.kernel, mesh=mesh,
        out_type=jax.ShapeDtypeStruct((n_tokens, _L), jnp.float32),
        scratch_types=[
            pltpu.VMEM((chunk, _E), jnp.float32),
            pltpu.VMEM((chunk, _L), jnp.float32),
        ],
        compiler_params=pltpu.CompilerParams(needs_layout_passes=False),
    )
    def sc_topk(keys_hbm, out_hbm, keys_v, out_v):
        wid = lax.axis_index("s") * _NC + lax.axis_index("c")
        base = wid * chunk
        pltpu.sync_copy(keys_hbm.at[pl.ds(base, chunk)], keys_v)

        def sortd(v):
            return lax.rev(lax.sort(v, dimension=0), (0,))

        def body(t, carry):
            s0 = sortd(keys_v[t, pl.ds(0, _L)])
            s1 = sortd(keys_v[t, pl.ds(_L, _L)])
            s2 = sortd(keys_v[t, pl.ds(2 * _L, _L)])
            s3 = sortd(keys_v[t, pl.ds(3 * _L, _L)])
            m01 = sortd(jnp.maximum(s0, lax.rev(s1, (0,))))
            m23 = sortd(jnp.maximum(s2, lax.rev(s3, (0,))))
            out_v[t, :] = sortd(jnp.maximum(m01, lax.rev(m23, (0,))))
            return carry

        lax.fori_loop(0, chunk, body, 0)
        pltpu.sync_copy(out_v, out_hbm.at[pl.ds(base, chunk)])

    return sc_topk


def kernel(x, weight):
    n_tokens = x.shape[0]
    keys = _tc_scores(x, weight)
    top = _make_sc_topk(n_tokens)(keys)[:, :_K]
    bits = jax.lax.bitcast_convert_type(top, jnp.int32)
    wout = jax.lax.bitcast_convert_type(bits & ~(_E - 1), jnp.float32)
    iout = (_E - 1) - (bits & (_E - 1))
    return wout, iout
